# baseline (speedup 1.0000x reference)
import numpy as np
import jax

jax.config.update("jax_enable_x64", True)
import jax.numpy as jnp
from jax.experimental import pallas as pl

_INPUT_DIM = 64
_NUM_PROJ = 16
_NUM_BINS = 9
_BLOCK = 8000

_W4 = np.zeros((_NUM_PROJ, 4), np.float32)
for _i in range(_NUM_PROJ):
    _W4[_i, _i // 4] = 10.0 ** (3 - _i % 4)


def _lsh_block(x_ref, proj_ref, w4_ref, out_ref):
    xb = x_ref[:]
    s = jnp.sum(xb * xb, axis=1, keepdims=True)
    nrm = jnp.maximum(jnp.sqrt(s), 1e-12)
    xn = (xb / nrm).astype(jnp.bfloat16)
    h = jnp.dot(xn, proj_ref[:], preferred_element_type=jnp.float32)
    d = jnp.clip(jnp.ceil(h * 4.5), -4.0, 5.0)
    codes = jnp.dot(d, w4_ref[:], preferred_element_type=jnp.float32) + 4444.0
    out_ref[:] = codes.astype(jnp.int32)


def kernel(x, projection_mat):
    n = x.shape[0]
    proj_bf16 = projection_mat.astype(jnp.bfloat16)
    grid = (n // _BLOCK,)
    groups = pl.pallas_call(
        _lsh_block,
        grid=grid,
        in_specs=[
            pl.BlockSpec((_BLOCK, _INPUT_DIM), lambda i: (i, i * 0)),
            pl.BlockSpec((_INPUT_DIM, _NUM_PROJ), lambda i: (i * 0, i * 0)),
            pl.BlockSpec((_NUM_PROJ, 4), lambda i: (i * 0, i * 0)),
        ],
        out_specs=pl.BlockSpec((_BLOCK, 4), lambda i: (i, i * 0)),
        out_shape=jax.ShapeDtypeStruct((n, 4), jnp.int32),
    )(x, proj_bf16, jnp.asarray(_W4))
    g = groups.astype(jnp.int64)
    return ((g[:, 0] * 10000 + g[:, 1]) * 10000 + g[:, 2]) * 10000 + g[:, 3]


# speedup vs baseline: 1.6487x; 1.6487x over previous
import numpy as np
import jax

jax.config.update("jax_enable_x64", True)
import jax.numpy as jnp
from jax.experimental import pallas as pl

_INPUT_DIM = 64
_NUM_PROJ = 16
_NUM_BINS = 9
_BLOCK = 4000

_W8 = np.zeros((2 * _NUM_PROJ, 8), np.float32)
for _r in range(2):
    for _i in range(_NUM_PROJ):
        _g, _pos = divmod(_i, 4)
        _col = (_g % 2) * 4 + (_g // 2) * 2 + _r
        _W8[_r * _NUM_PROJ + _i, _col] = 10.0 ** (3 - _pos)

_E2 = np.kron(np.eye(2, dtype=np.float32), np.ones((64, 64), np.float32))


def _lsh_block(x_ref, e2_ref, p2_ref, w8_ref, hi_ref, lo_ref):
    xb = x_ref[:]
    s = jnp.dot(xb * xb, e2_ref[:], preferred_element_type=jnp.float32)
    nrm = jnp.maximum(jnp.sqrt(s), 1e-12)
    xn = (xb / nrm).astype(jnp.bfloat16)
    h = jnp.dot(xn, p2_ref[:], preferred_element_type=jnp.float32)
    d = jnp.clip(jnp.ceil(h * 4.5), -4.0, 5.0)
    g8 = jnp.dot(d, w8_ref[:], preferred_element_type=jnp.float32) + 4444.0
    a = g8[:, 0:4].astype(jnp.int32)
    b = g8[:, 4:8].astype(jnp.int32)
    hilo = a * 10000 + b
    hi_ref[:] = hilo[:, 0:2]
    lo_ref[:] = hilo[:, 2:4]


def kernel(x, projection_mat):
    n = x.shape[0]
    x2 = x.reshape(n // 2, 2 * _INPUT_DIM)
    pb = projection_mat.astype(jnp.bfloat16)
    p2 = jnp.zeros((2 * _INPUT_DIM, 2 * _NUM_PROJ), jnp.bfloat16)
    p2 = p2.at[:_INPUT_DIM, :_NUM_PROJ].set(pb)
    p2 = p2.at[_INPUT_DIM:, _NUM_PROJ:].set(pb)
    grid = (x2.shape[0] // _BLOCK,)
    hi, lo = pl.pallas_call(
        _lsh_block,
        grid=grid,
        in_specs=[
            pl.BlockSpec((_BLOCK, 2 * _INPUT_DIM), lambda i: (i, i * 0)),
            pl.BlockSpec((2 * _INPUT_DIM, 2 * _INPUT_DIM), lambda i: (i * 0, i * 0)),
            pl.BlockSpec((2 * _INPUT_DIM, 2 * _NUM_PROJ), lambda i: (i * 0, i * 0)),
            pl.BlockSpec((2 * _NUM_PROJ, 8), lambda i: (i * 0, i * 0)),
        ],
        out_specs=[
            pl.BlockSpec((_BLOCK, 2), lambda i: (i, i * 0)),
            pl.BlockSpec((_BLOCK, 2), lambda i: (i, i * 0)),
        ],
        out_shape=[
            jax.ShapeDtypeStruct((n // 2, 2), jnp.int32),
            jax.ShapeDtypeStruct((n // 2, 2), jnp.int32),
        ],
    )(x2, jnp.asarray(_E2), p2, jnp.asarray(_W8))
    hi64 = hi.reshape(n).astype(jnp.int64)
    lo64 = lo.reshape(n).astype(jnp.int64)
    return hi64 * (10**8) + lo64


# speedup vs baseline: 3.4630x; 2.1004x over previous
import numpy as np
import jax

jax.config.update("jax_enable_x64", True)
import jax.numpy as jnp
from jax.experimental import pallas as pl

_INPUT_DIM = 64
_NUM_PROJ = 16
_NUM_BINS = 9
_S = 1000
_C = 4

_W8 = np.zeros((2 * _NUM_PROJ, 8), np.float32)
for _r in range(2):
    for _i in range(_NUM_PROJ):
        _g, _pos = divmod(_i, 4)
        _col = (_g % 2) * 4 + (_g // 2) * 2 + _r
        _W8[_r * _NUM_PROJ + _i, _col] = 10.0 ** (3 - _pos)

_E2 = np.kron(np.eye(2, dtype=np.float32), np.ones((64, 64), np.float32))


def _lsh_block(*refs):
    xs = refs[:2 * _C]
    e2_ref, p2_ref, w8_ref, hi_ref, lo_ref = refs[2 * _C:]
    x2 = jnp.concatenate(
        [jnp.concatenate([xs[2 * c][:], xs[2 * c + 1][:]], axis=1)
         for c in range(_C)], axis=0)
    s = jnp.dot(x2 * x2, e2_ref[:], preferred_element_type=jnp.float32,
                precision=jax.lax.Precision.HIGHEST)
    nrm = jnp.maximum(jnp.sqrt(s), 1e-12)
    xn = (x2 / nrm).astype(jnp.bfloat16)
    h = jnp.dot(xn, p2_ref[:], preferred_element_type=jnp.float32)
    d = jnp.clip(jnp.ceil(h * 4.5), -4.0, 5.0)
    g8 = jnp.dot(d, w8_ref[:], preferred_element_type=jnp.float32) + 4444.0
    g8t = g8.T
    a = g8t[0:4, :].astype(jnp.int32)
    b = g8t[4:8, :].astype(jnp.int32)
    hilo = a * 10000 + b
    hi_ref[:] = jnp.concatenate(
        [hilo[0:2, c * _S:(c + 1) * _S] for c in range(_C)], axis=0)
    lo_ref[:] = jnp.concatenate(
        [hilo[2:4, c * _S:(c + 1) * _S] for c in range(_C)], axis=0)


def kernel(x, projection_mat):
    n = x.shape[0]
    nsteps = n // (2 * _S * _C)
    pb = projection_mat.astype(jnp.bfloat16)
    p2 = jnp.zeros((2 * _INPUT_DIM, 2 * _NUM_PROJ), jnp.bfloat16)
    p2 = p2.at[:_INPUT_DIM, :_NUM_PROJ].set(pb)
    p2 = p2.at[_INPUT_DIM:, _NUM_PROJ:].set(pb)

    def slice_spec(c):
        return pl.BlockSpec((_S, _INPUT_DIM), lambda i, c=c: (2 * _C * i + c, i * 0))

    hi, lo = pl.pallas_call(
        _lsh_block,
        grid=(nsteps,),
        in_specs=(
            [slice_spec(c) for c in range(2 * _C)]
            + [
                pl.BlockSpec((2 * _INPUT_DIM, 2 * _INPUT_DIM), lambda i: (i * 0, i * 0)),
                pl.BlockSpec((2 * _INPUT_DIM, 2 * _NUM_PROJ), lambda i: (i * 0, i * 0)),
                pl.BlockSpec((2 * _NUM_PROJ, 8), lambda i: (i * 0, i * 0)),
            ]
        ),
        out_specs=[
            pl.BlockSpec((2 * _C, _S), lambda i: (i, i * 0)),
            pl.BlockSpec((2 * _C, _S), lambda i: (i, i * 0)),
        ],
        out_shape=[
            jax.ShapeDtypeStruct((n // _S // 2 * 2, _S), jnp.int32),
            jax.ShapeDtypeStruct((n // _S // 2 * 2, _S), jnp.int32),
        ],
    )(*([x] * (2 * _C)), jnp.asarray(_E2), p2, jnp.asarray(_W8))
    hi64 = hi.reshape(n).astype(jnp.int64)
    lo64 = lo.reshape(n).astype(jnp.int64)
    return hi64 * (10**8) + lo64


# speedup vs baseline: 3.5097x; 1.0135x over previous
import numpy as np
import jax

jax.config.update("jax_enable_x64", True)
import jax.numpy as jnp
from jax.experimental import pallas as pl

_INPUT_DIM = 64
_NUM_PROJ = 16
_NUM_BINS = 9
_S = 1000
_C = 4

_W8 = np.zeros((2 * _NUM_PROJ, 8), np.float32)
for _r in range(2):
    for _i in range(_NUM_PROJ):
        _g, _pos = divmod(_i, 4)
        _col = (_g % 2) * 4 + (_g // 2) * 2 + _r
        _W8[_r * _NUM_PROJ + _i, _col] = 10.0 ** (3 - _pos)

_E2 = np.kron(np.eye(2, dtype=np.float32), np.ones((64, 64), np.float32))


def _lsh_block(x_ref, e2_ref, p2_ref, w8_ref, hi_ref, lo_ref):
    x8 = x_ref[:]
    x2 = jnp.concatenate(
        [jnp.concatenate([x8[2 * c * _S:(2 * c + 1) * _S],
                          x8[(2 * c + 1) * _S:(2 * c + 2) * _S]], axis=1)
         for c in range(_C)], axis=0)
    s = jnp.dot(x2 * x2, e2_ref[:], preferred_element_type=jnp.float32,
                precision=jax.lax.Precision.HIGHEST)
    nrm = jnp.maximum(jnp.sqrt(s), 1e-12)
    xn = (x2 / nrm).astype(jnp.bfloat16)
    h = jnp.dot(xn, p2_ref[:], preferred_element_type=jnp.float32)
    d = jnp.clip(jnp.ceil(h * 4.5), -4.0, 5.0)
    g8 = jnp.dot(d, w8_ref[:], preferred_element_type=jnp.float32) + 4444.0
    g8t = g8.T
    a = g8t[0:4, :].astype(jnp.int32)
    b = g8t[4:8, :].astype(jnp.int32)
    hilo = a * 10000 + b
    hi_ref[:] = jnp.concatenate(
        [hilo[0:2, c * _S:(c + 1) * _S] for c in range(_C)], axis=0)
    lo_ref[:] = jnp.concatenate(
        [hilo[2:4, c * _S:(c + 1) * _S] for c in range(_C)], axis=0)


def kernel(x, projection_mat):
    n = x.shape[0]
    nsteps = n // (2 * _S * _C)
    pb = projection_mat.astype(jnp.bfloat16)
    p2 = jnp.zeros((2 * _INPUT_DIM, 2 * _NUM_PROJ), jnp.bfloat16)
    p2 = p2.at[:_INPUT_DIM, :_NUM_PROJ].set(pb)
    p2 = p2.at[_INPUT_DIM:, _NUM_PROJ:].set(pb)

    hi, lo = pl.pallas_call(
        _lsh_block,
        grid=(nsteps,),
        in_specs=[
            pl.BlockSpec((2 * _C * _S, _INPUT_DIM), lambda i: (i, i * 0)),
            pl.BlockSpec((2 * _INPUT_DIM, 2 * _INPUT_DIM), lambda i: (i * 0, i * 0)),
            pl.BlockSpec((2 * _INPUT_DIM, 2 * _NUM_PROJ), lambda i: (i * 0, i * 0)),
            pl.BlockSpec((2 * _NUM_PROJ, 8), lambda i: (i * 0, i * 0)),
        ],
        out_specs=[
            pl.BlockSpec((2 * _C, _S), lambda i: (i, i * 0)),
            pl.BlockSpec((2 * _C, _S), lambda i: (i, i * 0)),
        ],
        out_shape=[
            jax.ShapeDtypeStruct((n // _S // 2 * 2, _S), jnp.int32),
            jax.ShapeDtypeStruct((n // _S // 2 * 2, _S), jnp.int32),
        ],
    )(x, jnp.asarray(_E2), p2, jnp.asarray(_W8))
    hi64 = hi.reshape(n).astype(jnp.int64)
    lo64 = lo.reshape(n).astype(jnp.int64)
    return hi64 * (10**8) + lo64


# speedup vs baseline: 20.6052x; 5.8709x over previous
import numpy as np
import jax

jax.config.update("jax_enable_x64", True)
import jax.numpy as jnp
from jax.experimental import pallas as pl

_INPUT_DIM = 64
_NUM_PROJ = 16
_NUM_BINS = 9
_L = 16384
_OUTW = 2048

_W4 = np.zeros((4, _NUM_PROJ), np.float32)
for _i in range(_NUM_PROJ):
    _W4[_i // 4, _i] = 10.0 ** (3 - _i % 4)


def _lsh_block(xt_ref, pt_ref, w4_ref, hi_ref, lo_ref):
    xt = xt_ref[:]
    s = jnp.sum(xt * xt, axis=0, keepdims=True)
    nrm = jnp.maximum(jnp.sqrt(s), 1e-12)
    xn = (xt / nrm).astype(jnp.bfloat16)
    h = jnp.dot(pt_ref[:], xn, preferred_element_type=jnp.float32)
    d = jnp.clip(jnp.ceil(h * 4.5), -4.0, 5.0)
    g4 = jnp.dot(w4_ref[:], d, preferred_element_type=jnp.float32) + 4444.0
    hi = g4[0:1, :].astype(jnp.int32) * 10000 + g4[1:2, :].astype(jnp.int32)
    lo = g4[2:3, :].astype(jnp.int32) * 10000 + g4[3:4, :].astype(jnp.int32)
    hi_ref[:] = jnp.concatenate(
        [hi[:, c * _OUTW:(c + 1) * _OUTW] for c in range(8)], axis=0)
    lo_ref[:] = jnp.concatenate(
        [lo[:, c * _OUTW:(c + 1) * _OUTW] for c in range(8)], axis=0)


def kernel(x, projection_mat):
    n = x.shape[0]
    nsteps = -(-n // _L)
    xt = jnp.swapaxes(x, 0, 1)
    pt = jnp.swapaxes(projection_mat, 0, 1).astype(jnp.bfloat16)
    hi, lo = pl.pallas_call(
        _lsh_block,
        grid=(nsteps,),
        in_specs=[
            pl.BlockSpec((_INPUT_DIM, _L), lambda i: (i * 0, i)),
            pl.BlockSpec((_NUM_PROJ, _INPUT_DIM), lambda i: (i * 0, i * 0)),
            pl.BlockSpec((4, _NUM_PROJ), lambda i: (i * 0, i * 0)),
        ],
        out_specs=[
            pl.BlockSpec((8, _OUTW), lambda i: (i, i * 0)),
            pl.BlockSpec((8, _OUTW), lambda i: (i, i * 0)),
        ],
        out_shape=[
            jax.ShapeDtypeStruct((8 * nsteps, _OUTW), jnp.int32),
            jax.ShapeDtypeStruct((8 * nsteps, _OUTW), jnp.int32),
        ],
    )(xt, pt, jnp.asarray(_W4))
    hi64 = hi.reshape(-1)[:n].astype(jnp.int64)
    lo64 = lo.reshape(-1)[:n].astype(jnp.int64)
    return hi64 * (10**8) + lo64


# speedup vs baseline: 36.3895x; 1.7660x over previous
import numpy as np
import jax

jax.config.update("jax_enable_x64", True)
import jax.numpy as jnp
from jax.experimental import pallas as pl

_INPUT_DIM = 64
_NUM_PROJ = 16
_NUM_BINS = 9
_L = 32768
_OUTW = 2048

_W4 = np.zeros((4, _NUM_PROJ), np.float32)
for _i in range(_NUM_PROJ):
    _W4[_i // 4, _i] = 10.0 ** (3 - _i % 4)


def _lsh_block(xt_ref, pt_ref, w4_ref, hi_ref, lo_ref):
    xt = xt_ref[:]
    s = jnp.sum(xt * xt, axis=0, keepdims=True)
    nrm = jnp.maximum(jnp.sqrt(s), 1e-12)
    xn = (xt / nrm).astype(jnp.bfloat16)
    h = jnp.dot(pt_ref[:], xn, preferred_element_type=jnp.float32)
    d = jnp.clip(jnp.ceil(h * 4.5), -4.0, 5.0)
    g4 = jnp.dot(w4_ref[:], d, preferred_element_type=jnp.float32) + 4444.0
    hi = g4[0:1, :].astype(jnp.int32) * 10000 + g4[1:2, :].astype(jnp.int32)
    lo = g4[2:3, :].astype(jnp.int32) * 10000 + g4[3:4, :].astype(jnp.int32)
    hi_ref[:] = jnp.concatenate(
        [hi[:, c * _OUTW:(c + 1) * _OUTW] for c in range(8)], axis=0)
    lo_ref[:] = jnp.concatenate(
        [lo[:, c * _OUTW:(c + 1) * _OUTW] for c in range(8)], axis=0)


def kernel(x, projection_mat):
    n = x.shape[0]
    nsteps = -(-n // _L)
    xt = jnp.swapaxes(x, 0, 1)
    pt = jnp.swapaxes(projection_mat, 0, 1).astype(jnp.bfloat16)
    hi, lo = pl.pallas_call(
        _lsh_block,
        grid=(nsteps,),
        in_specs=[
            pl.BlockSpec((_INPUT_DIM, _L), lambda i: (i * 0, i)),
            pl.BlockSpec((_NUM_PROJ, _INPUT_DIM), lambda i: (i * 0, i * 0)),
            pl.BlockSpec((4, _NUM_PROJ), lambda i: (i * 0, i * 0)),
        ],
        out_specs=[
            pl.BlockSpec((8, _OUTW), lambda i: (i, i * 0)),
            pl.BlockSpec((8, _OUTW), lambda i: (i, i * 0)),
        ],
        out_shape=[
            jax.ShapeDtypeStruct((8 * nsteps, _OUTW), jnp.int32),
            jax.ShapeDtypeStruct((8 * nsteps, _OUTW), jnp.int32),
        ],
    )(xt, pt, jnp.asarray(_W4))
    hi64 = hi.reshape(-1)[:n].astype(jnp.int64)
    lo64 = lo.reshape(-1)[:n].astype(jnp.int64)
    return hi64 * (10**8) + lo64
